# baseline (speedup 1.0000x reference)
import jax
import jax.numpy as jnp
from jax import lax
from jax.experimental import pallas as pl
from jax.experimental.pallas import tpu as pltpu
from jax.experimental.pallas import tpu_sc as plsc

N, E, D = 10000, 320000, 128
NC, NS, L = 2, 16, 16
H = D // NC
EPT = E // NS
C = 80
NCHUNK = EPT // C
R = 5
RPT = 632
NP = NS * RPT
ZB = 104


def _sc_body(fcol, srco, dst, w, wb, part_out, deg_out, *scr):
    idx = scr[0:R]
    dstv = scr[R:2 * R]
    wv = scr[2 * R:3 * R]
    wbv = scr[3 * R:4 * R]
    rows = scr[4 * R:5 * R]
    zrow, zdeg, acc, dacc = scr[5 * R:5 * R + 4]
    semA = scr[5 * R + 4:5 * R + 4 + R]
    semG = scr[5 * R + 4 + R:5 * R + 4 + 2 * R]
    semS = scr[5 * R + 4 + 2 * R:5 * R + 4 + 3 * R]
    semW = scr[5 * R + 4 + 3 * R:5 * R + 4 + 4 * R]

    c = lax.axis_index("c")
    s = lax.axis_index("s")
    r0 = s * RPT
    base = s * EPT

    def zloop(i, carry):
        for j in range(H // L):
            zrow[i, pl.ds(j * L, L)] = jnp.zeros((L,), jnp.float32)
        zdeg[i, :] = jnp.zeros((L,), jnp.float32)
        return carry
    lax.fori_loop(0, ZB, zloop, 0)
    for o in (0, ZB, 2 * ZB, 3 * ZB, 4 * ZB, 5 * ZB, RPT - ZB):
        pltpu.sync_copy(zrow, acc.at[pl.ds(r0 + o, ZB)])

    @pl.when(c == 0)
    def _():
        for o in (0, ZB, 2 * ZB, 3 * ZB, 4 * ZB, 5 * ZB, RPT - ZB):
            pltpu.sync_copy(zdeg, dacc.at[pl.ds(r0 + o, ZB)])
    plsc.subcore_barrier()

    def off_of(k):
        return pl.multiple_of(jnp.minimum(base + k * C, E - C), 8)

    def loads_issue(k, b):
        o = off_of(k)
        pltpu.async_copy(srco.at[c, pl.ds(o, C)], idx[b], semA[b])
        pltpu.async_copy(dst.at[pl.ds(o, C)], dstv[b], semA[b])
        pltpu.async_copy(w.at[pl.ds(o, C)], wv[b], semA[b])

        @pl.when(c == 0)
        def _():
            pltpu.async_copy(wb.at[pl.ds(o, C)], wbv[b], semA[b])

    def loads_wait(k, b):
        o = off_of(k)
        pltpu.make_async_copy(srco.at[c, pl.ds(o, C)], idx[b], semA[b]).wait()
        pltpu.make_async_copy(dst.at[pl.ds(o, C)], dstv[b], semA[b]).wait()
        pltpu.make_async_copy(w.at[pl.ds(o, C)], wv[b], semA[b]).wait()

        @pl.when(c == 0)
        def _():
            pltpu.make_async_copy(wb.at[pl.ds(o, C)], wbv[b], semA[b]).wait()

    def gather_issue(b):
        pltpu.async_copy(fcol.at[idx[b]], rows[b], semG[b])

    def gather_wait(b):
        pltpu.make_async_copy(fcol.at[idx[b]], rows[b], semG[b]).wait()

    def rscat_issue(b):
        pltpu.async_copy(rows[b], acc.at[dstv[b]], semS[b], add=True)

    def rscat_wait(b):
        pltpu.make_async_copy(rows[b], acc.at[dstv[b]], semS[b]).wait()

    def wscat_issue(b):
        pltpu.async_copy(wbv[b], dacc.at[dstv[b]], semW[b], add=True)

    def wscat_wait(b):
        pltpu.make_async_copy(wbv[b], dacc.at[dstv[b]], semW[b]).wait()

    def scale(b):
        def body(t, icarry):
            wvec = wv[b][pl.ds(t * L, L)]
            for r in range(L):
                wi = wvec[r]
                i = t * L + r
                for j in range(H // L):
                    rows[b][i, pl.ds(j * L, L)] = rows[b][i, pl.ds(j * L, L)] * wi
            return icarry
        lax.fori_loop(0, C // L, body, 0)

    def step(k, b):
        bp1, bp2, bm2 = (b + 1) % R, (b + 2) % R, (b + 3) % R
        loads_issue(k + 2, bp2)
        loads_wait(k + 1, bp1)
        gather_issue(bp1)
        gather_wait(b)
        scale(b)
        rscat_issue(b)

        @pl.when(c == 0)
        def _():
            wscat_issue(b)

    loads_issue(0, 0)
    loads_issue(1, 1)
    loads_wait(0, 0)
    gather_issue(0)
    for b in range(R):
        if b >= 2:
            rscat_wait((b - 2) % R)

            @pl.when(c == 0)
            def _(b=b):
                wscat_wait((b - 2) % R)
        step(b, b)

    def outer(g, carry):
        for b in range(R):
            k = R * g + b
            bm2 = (b + 3) % R
            rscat_wait(bm2)

            @pl.when(c == 0)
            def _(bm2=bm2):
                wscat_wait(bm2)
            step(k, b)
        return carry
    lax.fori_loop(1, NCHUNK // R, outer, 0)

    for kk in (NCHUNK - 2, NCHUNK - 1):
        rscat_wait(kk % R)

        @pl.when(c == 0)
        def _(kk=kk):
            wscat_wait(kk % R)
    gather_wait(NCHUNK % R)
    loads_wait(NCHUNK + 1, (NCHUNK + 1) % R)

    plsc.subcore_barrier()
    pltpu.sync_copy(acc.at[pl.ds(r0, RPT)], part_out.at[c, pl.ds(r0, RPT)])

    @pl.when(c == 0)
    def _():
        pltpu.sync_copy(dacc.at[pl.ds(r0, RPT)], deg_out.at[pl.ds(r0, RPT)])


@jax.jit
def _sc_aggregate(fcol, srco, dst, w, wb):
    mesh = plsc.VectorSubcoreMesh(core_axis_name="c", subcore_axis_name="s")
    scratch = (
        [pltpu.VMEM((C,), jnp.int32) for _ in range(R)]
        + [pltpu.VMEM((C,), jnp.int32) for _ in range(R)]
        + [pltpu.VMEM((C,), jnp.float32) for _ in range(R)]
        + [pltpu.VMEM((C, L), jnp.float32) for _ in range(R)]
        + [pltpu.VMEM((C, H), jnp.float32) for _ in range(R)]
        + [
            pltpu.VMEM((ZB, H), jnp.float32),
            pltpu.VMEM((ZB, L), jnp.float32),
            pltpu.VMEM_SHARED((NP, H), jnp.float32),
            pltpu.VMEM_SHARED((NP, L), jnp.float32),
        ]
        + [pltpu.SemaphoreType.DMA for _ in range(4 * R)]
    )
    return pl.kernel(
        _sc_body,
        out_type=(
            jax.ShapeDtypeStruct((NC, NP, H), jnp.float32),
            jax.ShapeDtypeStruct((NP, L), jnp.float32),
        ),
        mesh=mesh,
        compiler_params=pltpu.CompilerParams(use_tc_tiling_on_sc=False),
        scratch_types=scratch,
    )(fcol, srco, dst, w, wb)


BN = 1000


def _tc_body(feat_ref, p_ref, d_ref, ws_ref, wna_ref, wnb_ref,
             bs_ref, bo_ref, out_ref):
    deg = d_ref[:, 0:1]
    inv = jnp.float32(1.0) / (deg + jnp.float32(1e-8))
    ha = p_ref[0] * inv
    hb = p_ref[1] * inv
    dn = (((1,), (1,)), ((), ()))
    out_ref[...] = (
        lax.dot_general(feat_ref[...], ws_ref[...], dn,
                        preferred_element_type=jnp.float32)
        + lax.dot_general(ha, wna_ref[...], dn,
                          preferred_element_type=jnp.float32)
        + lax.dot_general(hb, wnb_ref[...], dn,
                          preferred_element_type=jnp.float32)
        + bs_ref[...] + bo_ref[...]
    )


@jax.jit
def _tc_combine(feat, part, deg, W_self, Wn_a, Wn_b, b_self, bias_out):
    grid = (N // BN,)
    return pl.pallas_call(
        _tc_body,
        grid=grid,
        in_specs=[
            pl.BlockSpec((BN, D), lambda i: (i, 0)),
            pl.BlockSpec((NC, BN, H), lambda i: (0, i, 0)),
            pl.BlockSpec((BN, L), lambda i: (i, 0)),
            pl.BlockSpec((D, D), lambda i: (0, 0)),
            pl.BlockSpec((D, H), lambda i: (0, 0)),
            pl.BlockSpec((D, H), lambda i: (0, 0)),
            pl.BlockSpec((1, D), lambda i: (0, 0)),
            pl.BlockSpec((1, D), lambda i: (0, 0)),
        ],
        out_specs=pl.BlockSpec((BN, D), lambda i: (i, 0)),
        out_shape=jax.ShapeDtypeStruct((N, D), jnp.float32),
    )(feat, part, deg, W_self, Wn_a, Wn_b, b_self, bias_out)


def kernel(feat, edge_index, edge_weight, W_self, b_self, W_neigh, bias_out):
    src = edge_index[0]
    dst = edge_index[1]
    w = edge_weight[:, 0]
    wb = jnp.broadcast_to(edge_weight, (E, L))
    fcol = jnp.concatenate([feat[:, :H], feat[:, H:]], axis=0)
    srco = jnp.stack([src, src + N])
    part, deg = _sc_aggregate(fcol, srco, dst, w, wb)
    return _tc_combine(feat, part, deg, W_self, W_neigh[:, :H], W_neigh[:, H:],
                       b_self.reshape(1, D), bias_out.reshape(1, D))


# speedup vs baseline: 1.0016x; 1.0016x over previous
import jax
import jax.numpy as jnp
from jax import lax
from jax.experimental import pallas as pl
from jax.experimental.pallas import tpu as pltpu
from jax.experimental.pallas import tpu_sc as plsc

N, E, D = 10000, 320000, 128
NC, NS, L = 2, 16, 16
H = D // NC
EPT = E // NS
C = 80
NCHUNK = EPT // C
R = 5
RPT = 632
NP = NS * RPT
ZB = 104


def _sc_body(fcol, srco, dst, w, wb, part_out, deg_out, *scr):
    idx = scr[0:R]
    dstv = scr[R:2 * R]
    wv = scr[2 * R:3 * R]
    wbv = scr[3 * R:4 * R]
    rows = scr[4 * R:5 * R]
    zrow, zdeg, acc, dacc = scr[5 * R:5 * R + 4]
    semA = scr[5 * R + 4:5 * R + 4 + R]
    semG = scr[5 * R + 4 + R:5 * R + 4 + 2 * R]
    semS = scr[5 * R + 4 + 2 * R:5 * R + 4 + 3 * R]
    semW = scr[5 * R + 4 + 3 * R:5 * R + 4 + 4 * R]

    c = lax.axis_index("c")
    s = lax.axis_index("s")
    r0 = s * RPT
    base = s * EPT

    def zloop(i, carry):
        for j in range(H // L):
            zrow[i, pl.ds(j * L, L)] = jnp.zeros((L,), jnp.float32)
        zdeg[i, :] = jnp.zeros((L,), jnp.float32)
        return carry
    lax.fori_loop(0, ZB, zloop, 0)
    for o in (0, ZB, 2 * ZB, 3 * ZB, 4 * ZB, 5 * ZB, RPT - ZB):
        pltpu.sync_copy(zrow, acc.at[pl.ds(r0 + o, ZB)])

    @pl.when(c == 0)
    def _():
        for o in (0, ZB, 2 * ZB, 3 * ZB, 4 * ZB, 5 * ZB, RPT - ZB):
            pltpu.sync_copy(zdeg, dacc.at[pl.ds(r0 + o, ZB)])
    plsc.subcore_barrier()

    def off_of(k):
        return pl.multiple_of(jnp.minimum(base + k * C, E - C), 8)

    def loads_issue(k, b):
        o = off_of(k)
        pltpu.async_copy(srco.at[c, pl.ds(o, C)], idx[b], semA[b])
        pltpu.async_copy(dst.at[pl.ds(o, C)], dstv[b], semA[b])
        pltpu.async_copy(w.at[pl.ds(o, C)], wv[b], semA[b])
        pltpu.async_copy(wb.at[pl.ds(o, C)], wbv[b], semA[b])

    def loads_wait(k, b):
        o = off_of(k)
        pltpu.make_async_copy(srco.at[c, pl.ds(o, C)], idx[b], semA[b]).wait()
        pltpu.make_async_copy(dst.at[pl.ds(o, C)], dstv[b], semA[b]).wait()
        pltpu.make_async_copy(w.at[pl.ds(o, C)], wv[b], semA[b]).wait()
        pltpu.make_async_copy(wb.at[pl.ds(o, C)], wbv[b], semA[b]).wait()

    def gather_issue(b):
        pltpu.async_copy(fcol.at[idx[b]], rows[b], semG[b])

    def gather_wait(b):
        pltpu.make_async_copy(fcol.at[idx[b]], rows[b], semG[b]).wait()

    def rscat_issue(b):
        pltpu.async_copy(rows[b], acc.at[dstv[b]], semS[b], add=True)

    def rscat_wait(b):
        pltpu.make_async_copy(rows[b], acc.at[dstv[b]], semS[b]).wait()

    def wscat_issue(b):
        pltpu.async_copy(wbv[b], dacc.at[dstv[b]], semW[b], add=True)

    def wscat_wait(b):
        pltpu.make_async_copy(wbv[b], dacc.at[dstv[b]], semW[b]).wait()

    def scale(b):
        def body(t, icarry):
            wvec = wv[b][pl.ds(t * L, L)]
            for r in range(L):
                wi = wvec[r]
                i = t * L + r
                for j in range(H // L):
                    rows[b][i, pl.ds(j * L, L)] = rows[b][i, pl.ds(j * L, L)] * wi
            return icarry
        lax.fori_loop(0, C // L, body, 0)

    def step(k, b):
        bp1, bp2, bm2 = (b + 1) % R, (b + 2) % R, (b + 3) % R
        loads_issue(k + 2, bp2)
        loads_wait(k + 1, bp1)
        gather_issue(bp1)
        gather_wait(b)
        scale(b)
        rscat_issue(b)

        @pl.when(c == 0)
        def _():
            wscat_issue(b)

    loads_issue(0, 0)
    loads_issue(1, 1)
    loads_wait(0, 0)
    gather_issue(0)
    for b in range(R):
        if b >= 2:
            rscat_wait((b - 2) % R)

            @pl.when(c == 0)
            def _(b=b):
                wscat_wait((b - 2) % R)
        step(b, b)

    def outer(g, carry):
        for b in range(R):
            k = R * g + b
            bm2 = (b + 3) % R
            rscat_wait(bm2)

            @pl.when(c == 0)
            def _(bm2=bm2):
                wscat_wait(bm2)
            step(k, b)
        return carry
    lax.fori_loop(1, NCHUNK // R, outer, 0)

    for kk in (NCHUNK - 2, NCHUNK - 1):
        rscat_wait(kk % R)

        @pl.when(c == 0)
        def _(kk=kk):
            wscat_wait(kk % R)
    gather_wait(NCHUNK % R)
    loads_wait(NCHUNK + 1, (NCHUNK + 1) % R)

    plsc.subcore_barrier()
    pltpu.sync_copy(acc.at[pl.ds(r0, RPT)], part_out.at[c, pl.ds(r0, RPT)])

    @pl.when(c == 0)
    def _():
        pltpu.sync_copy(dacc.at[pl.ds(r0, RPT)], deg_out.at[pl.ds(r0, RPT)])


@jax.jit
def _sc_aggregate(fcol, srco, dst, w, wb):
    mesh = plsc.VectorSubcoreMesh(core_axis_name="c", subcore_axis_name="s")
    scratch = (
        [pltpu.VMEM((C,), jnp.int32) for _ in range(R)]
        + [pltpu.VMEM((C,), jnp.int32) for _ in range(R)]
        + [pltpu.VMEM((C,), jnp.float32) for _ in range(R)]
        + [pltpu.VMEM((C, L), jnp.float32) for _ in range(R)]
        + [pltpu.VMEM((C, H), jnp.float32) for _ in range(R)]
        + [
            pltpu.VMEM((ZB, H), jnp.float32),
            pltpu.VMEM((ZB, L), jnp.float32),
            pltpu.VMEM_SHARED((NP, H), jnp.float32),
            pltpu.VMEM_SHARED((NP, L), jnp.float32),
        ]
        + [pltpu.SemaphoreType.DMA for _ in range(4 * R)]
    )
    return pl.kernel(
        _sc_body,
        out_type=(
            jax.ShapeDtypeStruct((NC, NP, H), jnp.float32),
            jax.ShapeDtypeStruct((NP, L), jnp.float32),
        ),
        mesh=mesh,
        compiler_params=pltpu.CompilerParams(use_tc_tiling_on_sc=False),
        scratch_types=scratch,
    )(fcol, srco, dst, w, wb)


BN = 1000


def _tc_body(feat_ref, p_ref, d_ref, ws_ref, wna_ref, wnb_ref,
             bs_ref, bo_ref, out_ref):
    deg = d_ref[:, 0:1]
    inv = jnp.float32(1.0) / (deg + jnp.float32(1e-8))
    ha = p_ref[0] * inv
    hb = p_ref[1] * inv
    dn = (((1,), (1,)), ((), ()))
    out_ref[...] = (
        lax.dot_general(feat_ref[...], ws_ref[...], dn,
                        preferred_element_type=jnp.float32)
        + lax.dot_general(ha, wna_ref[...], dn,
                          preferred_element_type=jnp.float32)
        + lax.dot_general(hb, wnb_ref[...], dn,
                          preferred_element_type=jnp.float32)
        + bs_ref[...] + bo_ref[...]
    )


@jax.jit
def _tc_combine(feat, part, deg, W_self, Wn_a, Wn_b, b_self, bias_out):
    grid = (N // BN,)
    return pl.pallas_call(
        _tc_body,
        grid=grid,
        in_specs=[
            pl.BlockSpec((BN, D), lambda i: (i, 0)),
            pl.BlockSpec((NC, BN, H), lambda i: (0, i, 0)),
            pl.BlockSpec((BN, L), lambda i: (i, 0)),
            pl.BlockSpec((D, D), lambda i: (0, 0)),
            pl.BlockSpec((D, H), lambda i: (0, 0)),
            pl.BlockSpec((D, H), lambda i: (0, 0)),
            pl.BlockSpec((1, D), lambda i: (0, 0)),
            pl.BlockSpec((1, D), lambda i: (0, 0)),
        ],
        out_specs=pl.BlockSpec((BN, D), lambda i: (i, 0)),
        out_shape=jax.ShapeDtypeStruct((N, D), jnp.float32),
    )(feat, part, deg, W_self, Wn_a, Wn_b, b_self, bias_out)


def kernel(feat, edge_index, edge_weight, W_self, b_self, W_neigh, bias_out):
    src = edge_index[0]
    dst = edge_index[1]
    w = edge_weight[:, 0]
    wb = jnp.broadcast_to(edge_weight, (E, L))
    fcol = jnp.concatenate([feat[:, :H], feat[:, H:]], axis=0)
    srco = jnp.stack([src, src + N])
    part, deg = _sc_aggregate(fcol, srco, dst, w, wb)
    return _tc_combine(feat, part, deg, W_self, W_neigh[:, :H], W_neigh[:, H:],
                       b_self.reshape(1, D), bias_out.reshape(1, D))


# speedup vs baseline: 1.5260x; 1.5236x over previous
import jax
import jax.numpy as jnp
from jax import lax
from jax.experimental import pallas as pl
from jax.experimental.pallas import tpu as pltpu
from jax.experimental.pallas import tpu_sc as plsc

N, E, D = 10000, 320000, 128
NC, NS, L = 2, 16, 16
H = D // NC
EPT = E // NS
C = 80
NCHUNK = EPT // C
R = 5
RPT = 632
NP = NS * RPT
ZB = 104


def _sc_body(fcol, srco, dst, w, part_out, deg_out, *scr):
    idx = scr[0:R]
    dstv = scr[R:2 * R]
    wv = scr[2 * R:3 * R]
    wrow = scr[3 * R:4 * R]
    rows = scr[4 * R:5 * R]
    zrow, zdeg, acc, dacc = scr[5 * R:5 * R + 4]
    semA = scr[5 * R + 4:5 * R + 4 + R]
    semG = scr[5 * R + 4 + R:5 * R + 4 + 2 * R]
    semS = scr[5 * R + 4 + 2 * R:5 * R + 4 + 3 * R]
    semW = scr[5 * R + 4 + 3 * R:5 * R + 4 + 4 * R]

    c = lax.axis_index("c")
    s = lax.axis_index("s")
    r0 = s * RPT
    base = s * EPT

    def zloop(i, carry):
        for j in range(H // L):
            zrow[i, pl.ds(j * L, L)] = jnp.zeros((L,), jnp.float32)
        zdeg[i, :] = jnp.zeros((L,), jnp.float32)
        return carry
    lax.fori_loop(0, ZB, zloop, 0)
    for o in (0, ZB, 2 * ZB, 3 * ZB, 4 * ZB, 5 * ZB, RPT - ZB):
        pltpu.sync_copy(zrow, acc.at[pl.ds(r0 + o, ZB)])

    @pl.when(c == 0)
    def _():
        for o in (0, ZB, 2 * ZB, 3 * ZB, 4 * ZB, 5 * ZB, RPT - ZB):
            pltpu.sync_copy(zdeg, dacc.at[pl.ds(r0 + o, ZB)])
    plsc.subcore_barrier()

    def off_of(k):
        return pl.multiple_of(jnp.minimum(base + k * C, E - C), 8)

    def loads_issue(k, b):
        o = off_of(k)
        pltpu.async_copy(srco.at[c, pl.ds(o, C)], idx[b], semA[b])
        pltpu.async_copy(dst.at[pl.ds(o, C)], dstv[b], semA[b])
        pltpu.async_copy(w.at[pl.ds(o, C)], wv[b], semA[b])

    def loads_wait(k, b):
        o = off_of(k)
        pltpu.make_async_copy(srco.at[c, pl.ds(o, C)], idx[b], semA[b]).wait()
        pltpu.make_async_copy(dst.at[pl.ds(o, C)], dstv[b], semA[b]).wait()
        pltpu.make_async_copy(w.at[pl.ds(o, C)], wv[b], semA[b]).wait()

    def gather_issue(b):
        pltpu.async_copy(fcol.at[idx[b]], rows[b], semG[b])

    def gather_wait(b):
        pltpu.make_async_copy(fcol.at[idx[b]], rows[b], semG[b]).wait()

    def rscat_issue(b):
        pltpu.async_copy(rows[b], acc.at[dstv[b]], semS[b], add=True)

    def rscat_wait(b):
        pltpu.make_async_copy(rows[b], acc.at[dstv[b]], semS[b]).wait()

    def wscat_issue(b):
        pltpu.async_copy(wrow[b], dacc.at[dstv[b]], semW[b], add=True)

    def wscat_wait(b):
        pltpu.make_async_copy(wrow[b], dacc.at[dstv[b]], semW[b]).wait()

    def scale(b):
        def body(t, icarry):
            wvec = wv[b][pl.ds(t * L, L)]
            for r in range(L):
                wi = wvec[r]
                i = t * L + r
                for j in range(H // L):
                    rows[b][i, pl.ds(j * L, L)] = rows[b][i, pl.ds(j * L, L)] * wi
                wrow[b][i, :] = jnp.broadcast_to(wi, (L,))
            return icarry
        lax.fori_loop(0, C // L, body, 0)

    def step(k, b):
        bp1, bp2, bm2 = (b + 1) % R, (b + 2) % R, (b + 3) % R
        loads_issue(k + 2, bp2)
        loads_wait(k + 1, bp1)
        gather_issue(bp1)
        gather_wait(b)
        scale(b)
        rscat_issue(b)

        @pl.when(c == 0)
        def _():
            wscat_issue(b)

    loads_issue(0, 0)
    loads_issue(1, 1)
    loads_wait(0, 0)
    gather_issue(0)
    for b in range(R):
        if b >= 2:
            rscat_wait((b - 2) % R)

            @pl.when(c == 0)
            def _(b=b):
                wscat_wait((b - 2) % R)
        step(b, b)

    def outer(g, carry):
        for b in range(R):
            k = R * g + b
            bm2 = (b + 3) % R
            rscat_wait(bm2)

            @pl.when(c == 0)
            def _(bm2=bm2):
                wscat_wait(bm2)
            step(k, b)
        return carry
    lax.fori_loop(1, NCHUNK // R, outer, 0)

    for kk in (NCHUNK - 2, NCHUNK - 1):
        rscat_wait(kk % R)

        @pl.when(c == 0)
        def _(kk=kk):
            wscat_wait(kk % R)
    gather_wait(NCHUNK % R)
    loads_wait(NCHUNK + 1, (NCHUNK + 1) % R)

    plsc.subcore_barrier()
    pltpu.sync_copy(acc.at[pl.ds(r0, RPT)], part_out.at[c, pl.ds(r0, RPT)])

    @pl.when(c == 0)
    def _():
        pltpu.sync_copy(dacc.at[pl.ds(r0, RPT)], deg_out.at[pl.ds(r0, RPT)])


@jax.jit
def _sc_aggregate(fcol, srco, dst, w):
    mesh = plsc.VectorSubcoreMesh(core_axis_name="c", subcore_axis_name="s")
    scratch = (
        [pltpu.VMEM((C,), jnp.int32) for _ in range(R)]
        + [pltpu.VMEM((C,), jnp.int32) for _ in range(R)]
        + [pltpu.VMEM((C,), jnp.float32) for _ in range(R)]
        + [pltpu.VMEM((C, L), jnp.float32) for _ in range(R)]
        + [pltpu.VMEM((C, H), jnp.float32) for _ in range(R)]
        + [
            pltpu.VMEM((ZB, H), jnp.float32),
            pltpu.VMEM((ZB, L), jnp.float32),
            pltpu.VMEM_SHARED((NP, H), jnp.float32),
            pltpu.VMEM_SHARED((NP, L), jnp.float32),
        ]
        + [pltpu.SemaphoreType.DMA for _ in range(4 * R)]
    )
    return pl.kernel(
        _sc_body,
        out_type=(
            jax.ShapeDtypeStruct((NC, NP, H), jnp.float32),
            jax.ShapeDtypeStruct((NP, L), jnp.float32),
        ),
        mesh=mesh,
        compiler_params=pltpu.CompilerParams(use_tc_tiling_on_sc=False),
        scratch_types=scratch,
    )(fcol, srco, dst, w)


BN = 1000


def _tc_self_body(feat_ref, ws_ref, bs_ref, bo_ref, out_ref):
    dn = (((1,), (1,)), ((), ()))
    out_ref[...] = (
        lax.dot_general(feat_ref[...], ws_ref[...], dn,
                        preferred_element_type=jnp.float32)
        + bs_ref[...] + bo_ref[...]
    )


@jax.jit
def _tc_self(feat, W_self, b_self, bias_out):
    return pl.pallas_call(
        _tc_self_body,
        grid=(N // BN,),
        in_specs=[
            pl.BlockSpec((BN, D), lambda i: (i, 0)),
            pl.BlockSpec((D, D), lambda i: (0, 0)),
            pl.BlockSpec((1, D), lambda i: (0, 0)),
            pl.BlockSpec((1, D), lambda i: (0, 0)),
        ],
        out_specs=pl.BlockSpec((BN, D), lambda i: (i, 0)),
        out_shape=jax.ShapeDtypeStruct((N, D), jnp.float32),
    )(feat, W_self, b_self, bias_out)


def _tc_neigh_body(hs_ref, p_ref, d_ref, wna_ref, wnb_ref, out_ref):
    deg = d_ref[:, 0:1]
    inv = jnp.float32(1.0) / (deg + jnp.float32(1e-8))
    ha = p_ref[0] * inv
    hb = p_ref[1] * inv
    dn = (((1,), (1,)), ((), ()))
    out_ref[...] = (
        hs_ref[...]
        + lax.dot_general(ha, wna_ref[...], dn,
                          preferred_element_type=jnp.float32)
        + lax.dot_general(hb, wnb_ref[...], dn,
                          preferred_element_type=jnp.float32)
    )


@jax.jit
def _tc_neigh(hs, part, deg, Wn_a, Wn_b):
    return pl.pallas_call(
        _tc_neigh_body,
        grid=(N // BN,),
        in_specs=[
            pl.BlockSpec((BN, D), lambda i: (i, 0)),
            pl.BlockSpec((NC, BN, H), lambda i: (0, i, 0)),
            pl.BlockSpec((BN, L), lambda i: (i, 0)),
            pl.BlockSpec((D, H), lambda i: (0, 0)),
            pl.BlockSpec((D, H), lambda i: (0, 0)),
        ],
        out_specs=pl.BlockSpec((BN, D), lambda i: (i, 0)),
        out_shape=jax.ShapeDtypeStruct((N, D), jnp.float32),
    )(hs, part, deg, Wn_a, Wn_b)


def kernel(feat, edge_index, edge_weight, W_self, b_self, W_neigh, bias_out):
    src = edge_index[0]
    dst = edge_index[1]
    w = edge_weight[:, 0]
    fcol = feat.reshape(NC * N, H)
    srco = jnp.stack([2 * src, 2 * src + 1])
    part, deg = _sc_aggregate(fcol, srco, dst, w)
    hs = _tc_self(feat, W_self, b_self.reshape(1, D), bias_out.reshape(1, D))
    return _tc_neigh(hs, part, deg, W_neigh[:, :H], W_neigh[:, H:])


# speedup vs baseline: 1.6006x; 1.0489x over previous
import jax
import jax.numpy as jnp
from jax import lax
from jax.experimental import pallas as pl
from jax.experimental.pallas import tpu as pltpu
from jax.experimental.pallas import tpu_sc as plsc

N, E, D = 10000, 320000, 128
NC, NS, L = 2, 16, 16
H = D // NC
EPT = E // NS
C = 80
NCHUNK = EPT // C
R = 5
RPT = 632
NP = NS * RPT
ZB = 104


def _sc_body(fcol, srco, dst, w, part_out, deg_out, *scr):
    idx = scr[0:R]
    dstv = scr[R:2 * R]
    wv = scr[2 * R:3 * R]
    rows = scr[3 * R:4 * R]
    zrow, zdeg, acc, dacc = scr[4 * R:4 * R + 4]
    semA = scr[4 * R + 4:4 * R + 4 + R]
    semG = scr[4 * R + 4 + R:4 * R + 4 + 2 * R]
    semS = scr[4 * R + 4 + 2 * R:4 * R + 4 + 3 * R]
    semW = scr[4 * R + 4 + 3 * R:4 * R + 4 + 4 * R]

    c = lax.axis_index("c")
    s = lax.axis_index("s")
    r0 = s * RPT
    base = s * EPT

    def zloop(i, carry):
        for j in range(H // L):
            zrow[i, pl.ds(j * L, L)] = jnp.zeros((L,), jnp.float32)
        return carry
    lax.fori_loop(0, ZB, zloop, 0)
    for o2 in (0, L, 2 * L, 3 * L, 4 * L, 5 * L, ZB - L):
        zdeg[pl.ds(o2, L)] = jnp.zeros((L,), jnp.float32)
    for o in (0, ZB, 2 * ZB, 3 * ZB, 4 * ZB, 5 * ZB, RPT - ZB):
        pltpu.sync_copy(zrow, acc.at[pl.ds(r0 + o, ZB)])

    @pl.when(c == 0)
    def _():
        for o in (0, ZB, 2 * ZB, 3 * ZB, 4 * ZB, 5 * ZB, RPT - ZB):
            pltpu.sync_copy(zdeg, dacc.at[pl.ds(r0 + o, ZB)])
    plsc.subcore_barrier()

    def off_of(k):
        return pl.multiple_of(jnp.minimum(base + k * C, E - C), 8)

    def loads_issue(k, b):
        o = off_of(k)
        pltpu.async_copy(srco.at[c, pl.ds(o, C)], idx[b], semA[b])
        pltpu.async_copy(dst.at[pl.ds(o, C)], dstv[b], semA[b])
        pltpu.async_copy(w.at[pl.ds(o, C)], wv[b], semA[b])

    def loads_wait(k, b):
        o = off_of(k)
        pltpu.make_async_copy(srco.at[c, pl.ds(o, C)], idx[b], semA[b]).wait()
        pltpu.make_async_copy(dst.at[pl.ds(o, C)], dstv[b], semA[b]).wait()
        pltpu.make_async_copy(w.at[pl.ds(o, C)], wv[b], semA[b]).wait()

    def gather_issue(b):
        pltpu.async_copy(fcol.at[idx[b]], rows[b], semG[b])

    def gather_wait(b):
        pltpu.make_async_copy(fcol.at[idx[b]], rows[b], semG[b]).wait()

    def rscat_issue(b):
        pltpu.async_copy(rows[b], acc.at[dstv[b]], semS[b], add=True)

    def rscat_wait(b):
        pltpu.make_async_copy(rows[b], acc.at[dstv[b]], semS[b]).wait()

    def wscat_issue(b):
        pltpu.async_copy(wv[b], dacc.at[dstv[b]], semW[b], add=True)

    def wscat_wait(b):
        pltpu.make_async_copy(wv[b], dacc.at[dstv[b]], semW[b]).wait()

    def scale(b):
        def body(t, icarry):
            wvec = wv[b][pl.ds(t * L, L)]
            for r in range(L):
                wi = wvec[r]
                i = t * L + r
                for j in range(H // L):
                    rows[b][i, pl.ds(j * L, L)] = rows[b][i, pl.ds(j * L, L)] * wi
            return icarry
        lax.fori_loop(0, C // L, body, 0)

    def step(k, b):
        bp1, bp2, bm2 = (b + 1) % R, (b + 2) % R, (b + 3) % R
        loads_issue(k + 2, bp2)
        loads_wait(k + 1, bp1)
        gather_issue(bp1)
        gather_wait(b)
        scale(b)
        rscat_issue(b)

        @pl.when(c == 0)
        def _():
            wscat_issue(b)

    loads_issue(0, 0)
    loads_issue(1, 1)
    loads_wait(0, 0)
    gather_issue(0)
    for b in range(R):
        if b >= 2:
            rscat_wait((b - 2) % R)

            @pl.when(c == 0)
            def _(b=b):
                wscat_wait((b - 2) % R)
        step(b, b)

    def outer(g, carry):
        for b in range(R):
            k = R * g + b
            bm2 = (b + 3) % R
            rscat_wait(bm2)

            @pl.when(c == 0)
            def _(bm2=bm2):
                wscat_wait(bm2)
            step(k, b)
        return carry
    lax.fori_loop(1, NCHUNK // R, outer, 0)

    for kk in (NCHUNK - 2, NCHUNK - 1):
        rscat_wait(kk % R)

        @pl.when(c == 0)
        def _(kk=kk):
            wscat_wait(kk % R)
    gather_wait(NCHUNK % R)
    loads_wait(NCHUNK + 1, (NCHUNK + 1) % R)

    plsc.subcore_barrier()
    pltpu.sync_copy(acc.at[pl.ds(r0, RPT)], part_out.at[c, pl.ds(r0, RPT)])

    @pl.when(c == 0)
    def _():
        pltpu.sync_copy(dacc.at[pl.ds(r0, RPT)], deg_out.at[pl.ds(r0, RPT)])


@jax.jit
def _sc_aggregate(fcol, srco, dst, w):
    mesh = plsc.VectorSubcoreMesh(core_axis_name="c", subcore_axis_name="s")
    scratch = (
        [pltpu.VMEM((C,), jnp.int32) for _ in range(R)]
        + [pltpu.VMEM((C,), jnp.int32) for _ in range(R)]
        + [pltpu.VMEM((C,), jnp.float32) for _ in range(R)]
        + [pltpu.VMEM((C, H), jnp.float32) for _ in range(R)]
        + [
            pltpu.VMEM((ZB, H), jnp.float32),
            pltpu.VMEM((ZB,), jnp.float32),
            pltpu.VMEM_SHARED((NP, H), jnp.float32),
            pltpu.VMEM_SHARED((NP,), jnp.float32),
        ]
        + [pltpu.SemaphoreType.DMA for _ in range(4 * R)]
    )
    return pl.kernel(
        _sc_body,
        out_type=(
            jax.ShapeDtypeStruct((NC, NP, H), jnp.float32),
            jax.ShapeDtypeStruct((NP,), jnp.float32),
        ),
        mesh=mesh,
        compiler_params=pltpu.CompilerParams(use_tc_tiling_on_sc=False),
        scratch_types=scratch,
    )(fcol, srco, dst, w)


BN = 1000


def _tc_self_body(feat_ref, ws_ref, bs_ref, bo_ref, out_ref):
    dn = (((1,), (1,)), ((), ()))
    out_ref[...] = (
        lax.dot_general(feat_ref[...], ws_ref[...], dn,
                        preferred_element_type=jnp.float32)
        + bs_ref[...] + bo_ref[...]
    )


@jax.jit
def _tc_self(feat, W_self, b_self, bias_out):
    return pl.pallas_call(
        _tc_self_body,
        grid=(N // BN,),
        in_specs=[
            pl.BlockSpec((BN, D), lambda i: (i, 0)),
            pl.BlockSpec((D, D), lambda i: (0, 0)),
            pl.BlockSpec((1, D), lambda i: (0, 0)),
            pl.BlockSpec((1, D), lambda i: (0, 0)),
        ],
        out_specs=pl.BlockSpec((BN, D), lambda i: (i, 0)),
        out_shape=jax.ShapeDtypeStruct((N, D), jnp.float32),
    )(feat, W_self, b_self, bias_out)


def _tc_neigh_body(hs_ref, p_ref, d_ref, wna_ref, wnb_ref, out_ref):
    deg = d_ref[...]
    inv = jnp.float32(1.0) / (deg + jnp.float32(1e-8))
    ha = p_ref[0] * inv
    hb = p_ref[1] * inv
    dn = (((1,), (1,)), ((), ()))
    out_ref[...] = (
        hs_ref[...]
        + lax.dot_general(ha, wna_ref[...], dn,
                          preferred_element_type=jnp.float32)
        + lax.dot_general(hb, wnb_ref[...], dn,
                          preferred_element_type=jnp.float32)
    )


@jax.jit
def _tc_neigh(hs, part, deg, Wn_a, Wn_b):
    return pl.pallas_call(
        _tc_neigh_body,
        grid=(N // BN,),
        in_specs=[
            pl.BlockSpec((BN, D), lambda i: (i, 0)),
            pl.BlockSpec((NC, BN, H), lambda i: (0, i, 0)),
            pl.BlockSpec((BN, 1), lambda i: (i, 0)),
            pl.BlockSpec((D, H), lambda i: (0, 0)),
            pl.BlockSpec((D, H), lambda i: (0, 0)),
        ],
        out_specs=pl.BlockSpec((BN, D), lambda i: (i, 0)),
        out_shape=jax.ShapeDtypeStruct((N, D), jnp.float32),
    )(hs, part, deg, Wn_a, Wn_b)


def kernel(feat, edge_index, edge_weight, W_self, b_self, W_neigh, bias_out):
    src = edge_index[0]
    dst = edge_index[1]
    w = edge_weight[:, 0]
    fcol = feat.reshape(NC * N, H)
    srco = jnp.stack([2 * src, 2 * src + 1])
    part, deg = _sc_aggregate(fcol, srco, dst, w)
    hs = _tc_self(feat, W_self, b_self.reshape(1, D), bias_out.reshape(1, D))
    deg2 = deg[:N].reshape(N, 1)
    return _tc_neigh(hs, part, deg2, W_neigh[:, :H], W_neigh[:, H:])
